# baseline (speedup 1.0000x reference)
import functools

import jax
import jax.numpy as jnp
from jax import lax
from jax.experimental import pallas as pl
from jax.experimental.pallas import tpu as pltpu
from jax.experimental.pallas import tpu_sc as plsc

_NUM_PIECES = 32
_C = 128
_HW = 64
_BOARD_W = _C * _HW
_OUT_W = _NUM_PIECES * _C
_NW = 32
_K = 4


def _ctz32(x):
    low = x & (0 - x)
    is_top = low == jnp.int32(-2147483648)
    f = low.astype(jnp.float32)
    e = (plsc.bitcast(f, jnp.int32) >> 23) & 0xFF
    return jnp.where(is_top, jnp.int32(31), e - 127)


def _lane_bcast(v, lane):
    idx = jnp.full((16, 1), lane, jnp.int32)
    return lax.gather(
        v, idx,
        lax.GatherDimensionNumbers(
            offset_dims=(), collapsed_slice_dims=(0,), start_index_map=(0,)),
        (1,), mode=lax.GatherScatterMode.PROMISE_IN_BOUNDS)


def _sc_body(board_hbm, ids_hbm, out_hbm, buf, idsv, outv, occ_lo, occ_hi):
    B = board_hbm.shape[0] // _BOARD_W
    bpw = B // _NW
    nch = bpw // _K
    cid = lax.axis_index("c")
    sid = lax.axis_index("s")
    wid = sid * 2 + cid
    base = wid * bpw
    iota = lax.iota(jnp.int32, 16)
    zf16 = jnp.zeros((16,), jnp.float32)
    zi16 = jnp.zeros((16,), jnp.int32)

    buf[pl.ds(_K * _BOARD_W, 16)] = zf16

    def chunk_body(ch, carry):
        b0 = base + ch * _K
        pltpu.sync_copy(board_hbm.at[pl.ds(b0 * _BOARD_W, _K * _BOARD_W)],
                        buf.at[pl.ds(0, _K * _BOARD_W)])
        pltpu.sync_copy(ids_hbm.at[pl.ds(b0 * _HW, _K * _HW)], idsv)

        def board_body(k, carry2):
            for seg in range(3):
                occ_lo[pl.ds(16 * seg, 16)] = zi16
                occ_hi[pl.ds(16 * seg, 16)] = zi16
            for j in range(4):
                idv = idsv[pl.ds(k * _HW + 16 * j, 16)]
                bits = jnp.int32(1) << (iota + (16 if j % 2 else 0))
                plsc.addupdate_scatter(occ_lo if j < 2 else occ_hi,
                                       [idv], bits)
            lo_a = plsc.load_gather(occ_lo, [iota + 1])
            hi_a = plsc.load_gather(occ_hi, [iota + 1])
            lo_b = plsc.load_gather(occ_lo, [iota + 17])
            hi_b = plsc.load_gather(occ_hi, [iota + 17])

            def first_of(lo, hi):
                return jnp.where(
                    lo != 0, _ctz32(lo),
                    jnp.where(hi != 0, _ctz32(hi) + 32, jnp.int32(_HW)))

            first_a = first_of(lo_a, hi_a)
            first_b = first_of(lo_b, hi_b)

            kboard = k * _BOARD_W
            kout = k * _OUT_W
            for t in range(_NUM_PIECES):
                fv = _lane_bcast(first_a if t < 16 else first_b, t % 16)
                absent = fv == _HW
                fvk = fv + kboard
                for c0 in range(0, _C, 16):
                    idxv = fvk + (c0 + iota) * _HW
                    g = plsc.load_gather(buf, [idxv])
                    g = jnp.where(absent, zf16, g)
                    outv[pl.ds(kout + t * _C + c0, 16)] = g
            return carry2

        lax.fori_loop(0, _K, board_body, 0)
        pltpu.sync_copy(outv.at[pl.ds(0, _K * _OUT_W)],
                        out_hbm.at[pl.ds(b0 * _OUT_W, _K * _OUT_W)])
        return carry

    lax.fori_loop(0, nch, chunk_body, 0)


def kernel(full_board_vector, piece_ids):
    B, C, H, W = full_board_vector.shape
    board_flat = full_board_vector.reshape(B * _BOARD_W)
    ids_flat = piece_ids.reshape(B * _HW)

    mesh = plsc.VectorSubcoreMesh(core_axis_name="c", subcore_axis_name="s")
    run = functools.partial(
        pl.kernel,
        out_type=jax.ShapeDtypeStruct((B * _OUT_W,), jnp.float32),
        mesh=mesh,
        compiler_params=pltpu.CompilerParams(needs_layout_passes=False),
        scratch_types=[
            pltpu.VMEM((_K * _BOARD_W + 16,), jnp.float32),
            pltpu.VMEM((_K * _HW,), jnp.int32),
            pltpu.VMEM((_K * _OUT_W,), jnp.float32),
            pltpu.VMEM((48,), jnp.int32),
            pltpu.VMEM((48,), jnp.int32),
        ],
    )(_sc_body)
    return run(board_flat, ids_flat).reshape(B, _NUM_PIECES, C)
